# Initial kernel scaffold; baseline (speedup 1.0000x reference)
import functools

import jax
import jax.numpy as jnp
from jax import lax
from jax.experimental import pallas as pl
from jax.experimental.pallas import tpu as pltpu
from jax.experimental.pallas import tpu_sc as plsc

N = 10000
FEAT = 256
HALF = 128
NOUT = 128
E = 160000

EB = 128
NBLK = 1280
EP = NBLK * EB
NSUB = 16
BPS = NBLK // NSUB
RPS = N // NSUB
RCH = 125
ACC_ROWS = N + 16

BM = 400
GRID = N // BM

_PREC = lax.Precision.HIGHEST

_mesh = plsc.VectorSubcoreMesh(core_axis_name="c", subcore_axis_name="s")


def _agg16_body(s16_hbm, srcb_hbm, dstb_hbm, out_hbm,
                idxs, idxd, rows, stage, acc, sem):
    cid = lax.axis_index("c")
    sid = lax.axis_index("s")

    @pl.when(cid == 0)
    def _():
        @pl.loop(0, RPS // RCH)
        def _(k):
            r0 = sid * RPS + k * RCH
            pltpu.sync_copy(s16_hbm.at[pl.ds(r0, RCH)], stage)
            pltpu.sync_copy(stage, acc.at[pl.ds(r0, RCH)])

        plsc.subcore_barrier()

        @pl.loop(0, BPS)
        def _(j):
            b = sid * BPS + j
            pltpu.sync_copy(srcb_hbm.at[b], idxs)
            pltpu.sync_copy(dstb_hbm.at[b], idxd)
            pltpu.async_copy(s16_hbm.at[idxs], rows, sem).wait()
            pltpu.sync_copy(rows, acc.at[idxd], add=True)

        plsc.subcore_barrier()

        @pl.loop(0, RPS // RCH)
        def _(k):
            r0 = sid * RPS + k * RCH
            pltpu.sync_copy(acc.at[pl.ds(r0, RCH)], stage)
            pltpu.sync_copy(stage, out_hbm.at[pl.ds(r0, RCH)])


@functools.partial(
    pl.kernel,
    mesh=_mesh,
    out_type=jax.ShapeDtypeStruct((N, 16), jnp.float32),
    scratch_types=[
        pltpu.VMEM((EB,), jnp.int32),
        pltpu.VMEM((EB,), jnp.int32),
        pltpu.VMEM((EB, 16), jnp.float32),
        pltpu.VMEM((RCH, 16), jnp.float32),
        pltpu.VMEM_SHARED((ACC_ROWS, 16), jnp.float32),
        pltpu.SemaphoreType.DMA,
    ],
)
def _agg16(*args):
    _agg16_body(*args)


def _agg128_body(x_hbm, srcb2_hbm, dstb_hbm, out_hbm,
                 idxs, idxd, rows, stage, acc, sem):
    cid = lax.axis_index("c")
    sid = lax.axis_index("s")
    off = cid * N

    @pl.loop(0, RPS // RCH)
    def _(k):
        r0 = sid * RPS + k * RCH
        pltpu.sync_copy(x_hbm.at[pl.ds(off + r0, RCH)], stage)
        pltpu.sync_copy(stage, acc.at[pl.ds(r0, RCH)])

    plsc.subcore_barrier()

    @pl.loop(0, BPS)
    def _(j):
        b = sid * BPS + j
        pltpu.sync_copy(srcb2_hbm.at[cid, b], idxs)
        pltpu.sync_copy(dstb_hbm.at[b], idxd)
        pltpu.async_copy(x_hbm.at[idxs], rows, sem).wait()
        pltpu.sync_copy(rows, acc.at[idxd], add=True)

    plsc.subcore_barrier()

    @pl.loop(0, RPS // RCH)
    def _(k):
        r0 = sid * RPS + k * RCH
        pltpu.sync_copy(acc.at[pl.ds(r0, RCH)], stage)
        pltpu.sync_copy(stage, out_hbm.at[cid, pl.ds(r0, RCH)])


@functools.partial(
    pl.kernel,
    mesh=_mesh,
    out_type=jax.ShapeDtypeStruct((2, N, HALF), jnp.float32),
    scratch_types=[
        pltpu.VMEM((EB,), jnp.int32),
        pltpu.VMEM((EB,), jnp.int32),
        pltpu.VMEM((EB, HALF), jnp.float32),
        pltpu.VMEM((RCH, HALF), jnp.float32),
        pltpu.VMEM_SHARED((ACC_ROWS, HALF), jnp.float32),
        pltpu.SemaphoreType.DMA,
    ],
)
def _agg128(*args):
    _agg128_body(*args)


def _stats_body(t_ref, w0_ref, b0_ref, stats_ref):
    i = pl.program_id(0)
    xp = jnp.maximum(t_ref[:, 0:1] * w0_ref[...] + b0_ref[...], 0.0)

    @pl.when(i == 0)
    def _():
        stats_ref[...] = jnp.zeros_like(stats_ref)

    stats_ref[0:1, :] += jnp.sum(xp, axis=0, keepdims=True)
    stats_ref[1:2, :] += jnp.sum(xp * xp, axis=0, keepdims=True)


def _norm_body(t_ref, stats_ref, w0_ref, b0_ref, g_ref, be_ref, al_ref, out_ref):
    xp = jnp.maximum(t_ref[:, 0:1] * w0_ref[...] + b0_ref[...], 0.0)
    m = stats_ref[0:1, :] * (1.0 / N)
    ex2 = stats_ref[1:2, :] * (1.0 / N)
    al = al_ref[...]
    var = ex2 - 2.0 * al * m * m + al * al * m * m
    y = g_ref[...] * ((xp - al * m) * lax.rsqrt(var + 1e-5)) + be_ref[...]
    out_ref[0] = y[:, :HALF]
    out_ref[1] = y[:, HALF:]


def _mlp_body(h_ref, x_ref, w1_ref, b1_ref, w2_ref, b2_ref, out_ref):
    h = jnp.concatenate([h_ref[0], h_ref[1]], axis=1)
    z = jnp.maximum(
        jnp.dot(h, w1_ref[...], preferred_element_type=jnp.float32,
                precision=_PREC) + b1_ref[...], 0.0)
    y = jnp.dot(z, w2_ref[...], preferred_element_type=jnp.float32,
                precision=_PREC) + b2_ref[...]
    x = jnp.concatenate([x_ref[0], x_ref[1]], axis=1)
    xn = x + jnp.maximum(y, 0.0)
    out_ref[0] = xn[:, :HALF]
    out_ref[1] = xn[:, HALF:]


def _final_body(x_ref, wf_ref, bf_ref, out_ref):
    x = jnp.concatenate([x_ref[0], x_ref[1]], axis=1)
    out_ref[...] = jnp.dot(x, wf_ref[...], preferred_element_type=jnp.float32,
                           precision=_PREC) + bf_ref[...]


def _row_spec(shape):
    return pl.BlockSpec(shape, lambda i: tuple(0 for _ in shape))


def _tc_stats(t16, w0row, b0row):
    return pl.pallas_call(
        _stats_body,
        grid=(GRID,),
        in_specs=[
            pl.BlockSpec((BM, 16), lambda i: (i, 0)),
            _row_spec((1, FEAT)),
            _row_spec((1, FEAT)),
        ],
        out_specs=pl.BlockSpec((8, FEAT), lambda i: (0, 0)),
        out_shape=jax.ShapeDtypeStruct((8, FEAT), jnp.float32),
    )(t16, w0row, b0row)


def _tc_norm(t16, stats, w0row, b0row, grow, berow, alrow):
    return pl.pallas_call(
        _norm_body,
        grid=(GRID,),
        in_specs=[
            pl.BlockSpec((BM, 16), lambda i: (i, 0)),
            _row_spec((8, FEAT)),
            _row_spec((1, FEAT)),
            _row_spec((1, FEAT)),
            _row_spec((1, FEAT)),
            _row_spec((1, FEAT)),
            _row_spec((1, FEAT)),
        ],
        out_specs=pl.BlockSpec((2, BM, HALF), lambda i: (0, i, 0)),
        out_shape=jax.ShapeDtypeStruct((2, N, HALF), jnp.float32),
    )(t16, stats, w0row, b0row, grow, berow, alrow)


def _tc_mlp(h2, x2, w1t, b1row, w2t, b2row):
    return pl.pallas_call(
        _mlp_body,
        grid=(GRID,),
        in_specs=[
            pl.BlockSpec((2, BM, HALF), lambda i: (0, i, 0)),
            pl.BlockSpec((2, BM, HALF), lambda i: (0, i, 0)),
            _row_spec((FEAT, FEAT)),
            _row_spec((1, FEAT)),
            _row_spec((FEAT, FEAT)),
            _row_spec((1, FEAT)),
        ],
        out_specs=pl.BlockSpec((2, BM, HALF), lambda i: (0, i, 0)),
        out_shape=jax.ShapeDtypeStruct((2, N, HALF), jnp.float32),
    )(h2, x2, w1t, b1row, w2t, b2row)


def _tc_final(x2, wft, bfrow):
    return pl.pallas_call(
        _final_body,
        grid=(GRID,),
        in_specs=[
            pl.BlockSpec((2, BM, HALF), lambda i: (0, i, 0)),
            _row_spec((FEAT, NOUT)),
            _row_spec((1, NOUT)),
        ],
        out_specs=pl.BlockSpec((BM, NOUT), lambda i: (i, 0)),
        out_shape=jax.ShapeDtypeStruct((N, NOUT), jnp.float32),
    )(x2, wft, bfrow)


def kernel(signals, edge_index, W0, b0, W1_0, b1_0, W2_0, b2_0, W1_1, b1_1,
           W2_1, W1_2, b1_2, W2_2, gn_gamma, gn_beta, gn_alpha, Wf, bf):
    src = edge_index[0].astype(jnp.int32)
    dst = edge_index[1].astype(jnp.int32)
    pad = EP - E
    srcp = jnp.concatenate([src, jnp.zeros((pad,), jnp.int32)]).reshape(NBLK, EB)
    dstp = jnp.concatenate([dst, jnp.full((pad,), N, jnp.int32)]).reshape(NBLK, EB)
    srcb2 = jnp.stack([srcp, srcp + N])

    s16 = jnp.broadcast_to(signals, (N, 16))

    w0row = W0.reshape(1, FEAT)
    b0row = b0.reshape(1, FEAT)
    grow = gn_gamma.reshape(1, FEAT)
    berow = gn_beta.reshape(1, FEAT)
    alrow = gn_alpha.reshape(1, FEAT)
    zrow = jnp.zeros((1, FEAT), jnp.float32)

    t16 = _agg16(s16, srcp, dstp)
    stats = _tc_stats(t16, w0row, b0row)
    x2 = _tc_norm(t16, stats, w0row, b0row, grow, berow, alrow)

    layer_params = [
        (W1_0.T, b1_0.reshape(1, FEAT), W2_0.T, b2_0.reshape(1, FEAT)),
        (W1_1.T, b1_1.reshape(1, FEAT), W2_1.T, zrow),
        (W1_2.T, b1_2.reshape(1, FEAT), W2_2.T, zrow),
    ]
    for w1t, b1row, w2t, b2row in layer_params:
        h2 = _agg128(x2.reshape(2 * N, HALF), srcb2, dstp)
        x2 = _tc_mlp(h2, x2, w1t, b1row, w2t, b2row)

    return _tc_final(x2, Wf.T, bf.reshape(1, NOUT))


# speedup vs baseline: 2.7259x; 2.7259x over previous
import dataclasses
import functools

import jax
import jax.numpy as jnp
from jax import lax
from jax.experimental import pallas as pl
from jax.experimental.pallas import tpu as pltpu
from jax.experimental.pallas import tpu_sc as plsc

N = 10000
FEAT = 256
HALF = 128
NOUT = 128
E = 160000

EB = 128
NBLK = 1280
EP = NBLK * EB
NSUB = 16
BPS = NBLK // NSUB
RPS = 624
RCH = 208
TAIL = N - NSUB * RPS
ACC_ROWS = N + 16
A1_ROWS = 10112
RED = 640

BM = 400
GRID = N // BM

_PREC = lax.Precision.HIGHEST

_mesh = plsc.VectorSubcoreMesh(core_axis_name="c", subcore_axis_name="s")

_sc_params = pltpu.CompilerParams()
if "needs_layout_passes" in pltpu.CompilerParams.__dataclass_fields__:
    _sc_params = dataclasses.replace(_sc_params, needs_layout_passes=False)


def _agg1_body(sigp_hbm, srcb_hbm, dstb_hbm, out_hbm,
               sig_v, idx3s, idx3d, acc_v, red_v, outacc, sh, sem):
    cid = lax.axis_index("c")
    sid = lax.axis_index("s")

    @pl.when(cid == 0)
    def _():
        pltpu.sync_copy(sigp_hbm, sig_v)
        pltpu.sync_copy(srcb_hbm.at[pl.ds(sid * BPS, BPS)], idx3s)
        pltpu.sync_copy(dstb_hbm.at[pl.ds(sid * BPS, BPS)], idx3d)

        @pl.loop(0, A1_ROWS // 16)
        def _(i):
            acc_v[pl.ds(i * 16, 16)] = jnp.zeros((16,), jnp.float32)

        @pl.loop(0, BPS)
        def _(j):
            @pl.loop(0, EB // 16)
            def _(k):
                sv = idx3s[j, 0, pl.ds(k * 16, 16)]
                dv = idx3d[j, 0, pl.ds(k * 16, 16)]
                vals = plsc.load_gather(sig_v, [sv])
                plsc.addupdate_scatter(acc_v, [dv], vals)

        pltpu.sync_copy(acc_v, sh.at[sid, 0])
        plsc.subcore_barrier()

        c0 = jnp.minimum(sid * RED, A1_ROWS - RED)
        pltpu.sync_copy(sh.at[:, :, pl.ds(c0, RED)], red_v)

        @pl.loop(0, RED // 16)
        def _(i):
            v = sig_v[pl.ds(c0 + i * 16, 16)]
            for k in range(NSUB):
                v = v + red_v[k, 0, pl.ds(i * 16, 16)]
            outacc[pl.ds(i * 16, 16)] = v

        pltpu.sync_copy(outacc, out_hbm.at[pl.ds(c0, RED)])


@functools.partial(
    pl.kernel,
    mesh=_mesh,
    out_type=jax.ShapeDtypeStruct((A1_ROWS,), jnp.float32),
    scratch_types=[
        pltpu.VMEM((A1_ROWS,), jnp.float32),
        pltpu.VMEM((BPS, 1, EB), jnp.int32),
        pltpu.VMEM((BPS, 1, EB), jnp.int32),
        pltpu.VMEM((A1_ROWS,), jnp.float32),
        pltpu.VMEM((NSUB, 1, RED), jnp.float32),
        pltpu.VMEM((RED,), jnp.float32),
        pltpu.VMEM_SHARED((NSUB, 1, A1_ROWS), jnp.float32),
        pltpu.SemaphoreType.DMA,
    ],
    compiler_params=_sc_params,
)
def _agg1(*args):
    _agg1_body(*args)


def _agg128_body(x_hbm, srcb2_hbm, dstb_hbm, out_hbm,
                 idxs, idxd, rows, stage, tstage, acc, sem):
    cid = lax.axis_index("c")
    sid = lax.axis_index("s")
    off = cid * N

    @pl.loop(0, RPS // RCH)
    def _(k):
        r0 = sid * RPS + k * RCH
        pltpu.sync_copy(x_hbm.at[pl.ds(off + r0, RCH)], stage)
        pltpu.sync_copy(stage, acc.at[pl.ds(r0, RCH)])

    @pl.when(sid == NSUB - 1)
    def _():
        pltpu.sync_copy(x_hbm.at[pl.ds(off + NSUB * RPS, TAIL)], tstage)
        pltpu.sync_copy(tstage, acc.at[pl.ds(NSUB * RPS, TAIL)])

    plsc.subcore_barrier()

    @pl.loop(0, BPS)
    def _(j):
        b = sid * BPS + j
        pltpu.sync_copy(srcb2_hbm.at[cid, b], idxs)
        pltpu.sync_copy(dstb_hbm.at[b], idxd)
        pltpu.async_copy(x_hbm.at[idxs.at[0]], rows, sem).wait()
        pltpu.sync_copy(rows, acc.at[idxd.at[0]], add=True)

    plsc.subcore_barrier()

    @pl.loop(0, RPS // RCH)
    def _(k):
        r0 = sid * RPS + k * RCH
        pltpu.sync_copy(acc.at[pl.ds(r0, RCH)], stage)
        pltpu.sync_copy(stage, out_hbm.at[cid, pl.ds(r0, RCH)])

    @pl.when(sid == NSUB - 1)
    def _():
        pltpu.sync_copy(acc.at[pl.ds(NSUB * RPS, TAIL)], tstage)
        pltpu.sync_copy(tstage, out_hbm.at[cid, pl.ds(NSUB * RPS, TAIL)])


@functools.partial(
    pl.kernel,
    mesh=_mesh,
    out_type=jax.ShapeDtypeStruct((2, N, HALF), jnp.float32),
    scratch_types=[
        pltpu.VMEM((1, EB), jnp.int32),
        pltpu.VMEM((1, EB), jnp.int32),
        pltpu.VMEM((EB, HALF), jnp.float32),
        pltpu.VMEM((RCH, HALF), jnp.float32),
        pltpu.VMEM((TAIL, HALF), jnp.float32),
        pltpu.VMEM_SHARED((ACC_ROWS, HALF), jnp.float32),
        pltpu.SemaphoreType.DMA,
    ],
)
def _agg128(*args):
    _agg128_body(*args)


def _stats_body(t_ref, w0_ref, b0_ref, stats_ref):
    i = pl.program_id(0)
    xp = jnp.maximum(t_ref[...] * w0_ref[...] + b0_ref[...], 0.0)

    @pl.when(i == 0)
    def _():
        stats_ref[...] = jnp.zeros_like(stats_ref)

    stats_ref[0:1, :] += jnp.sum(xp, axis=0, keepdims=True)
    stats_ref[1:2, :] += jnp.sum(xp * xp, axis=0, keepdims=True)


def _norm_body(t_ref, stats_ref, w0_ref, b0_ref, g_ref, be_ref, al_ref, out_ref):
    xp = jnp.maximum(t_ref[...] * w0_ref[...] + b0_ref[...], 0.0)
    m = stats_ref[0:1, :] * (1.0 / N)
    ex2 = stats_ref[1:2, :] * (1.0 / N)
    al = al_ref[...]
    var = ex2 - 2.0 * al * m * m + al * al * m * m
    y = g_ref[...] * ((xp - al * m) * lax.rsqrt(var + 1e-5)) + be_ref[...]
    out_ref[0] = y[:, :HALF]
    out_ref[1] = y[:, HALF:]


def _mlp_body(h_ref, x_ref, w1_ref, b1_ref, w2_ref, b2_ref, out_ref):
    h = jnp.concatenate([h_ref[0], h_ref[1]], axis=1)
    z = jnp.maximum(
        jnp.dot(h, w1_ref[...], preferred_element_type=jnp.float32,
                precision=_PREC) + b1_ref[...], 0.0)
    y = jnp.dot(z, w2_ref[...], preferred_element_type=jnp.float32,
                precision=_PREC) + b2_ref[...]
    x = jnp.concatenate([x_ref[0], x_ref[1]], axis=1)
    xn = x + jnp.maximum(y, 0.0)
    out_ref[0] = xn[:, :HALF]
    out_ref[1] = xn[:, HALF:]


def _final_body(x_ref, wf_ref, bf_ref, out_ref):
    x = jnp.concatenate([x_ref[0], x_ref[1]], axis=1)
    out_ref[...] = jnp.dot(x, wf_ref[...], preferred_element_type=jnp.float32,
                           precision=_PREC) + bf_ref[...]


def _row_spec(shape):
    return pl.BlockSpec(shape, lambda i: tuple(0 for _ in shape))


def _tc_stats(t1, w0row, b0row):
    return pl.pallas_call(
        _stats_body,
        grid=(GRID,),
        in_specs=[
            pl.BlockSpec((BM, 1), lambda i: (i, 0)),
            _row_spec((1, FEAT)),
            _row_spec((1, FEAT)),
        ],
        out_specs=pl.BlockSpec((8, FEAT), lambda i: (0, 0)),
        out_shape=jax.ShapeDtypeStruct((8, FEAT), jnp.float32),
    )(t1, w0row, b0row)


def _tc_norm(t1, stats, w0row, b0row, grow, berow, alrow):
    return pl.pallas_call(
        _norm_body,
        grid=(GRID,),
        in_specs=[
            pl.BlockSpec((BM, 1), lambda i: (i, 0)),
            _row_spec((8, FEAT)),
            _row_spec((1, FEAT)),
            _row_spec((1, FEAT)),
            _row_spec((1, FEAT)),
            _row_spec((1, FEAT)),
            _row_spec((1, FEAT)),
        ],
        out_specs=pl.BlockSpec((2, BM, HALF), lambda i: (0, i, 0)),
        out_shape=jax.ShapeDtypeStruct((2, N, HALF), jnp.float32),
    )(t1, stats, w0row, b0row, grow, berow, alrow)


def _tc_mlp(h2, x2, w1t, b1row, w2t, b2row):
    return pl.pallas_call(
        _mlp_body,
        grid=(GRID,),
        in_specs=[
            pl.BlockSpec((2, BM, HALF), lambda i: (0, i, 0)),
            pl.BlockSpec((2, BM, HALF), lambda i: (0, i, 0)),
            _row_spec((FEAT, FEAT)),
            _row_spec((1, FEAT)),
            _row_spec((FEAT, FEAT)),
            _row_spec((1, FEAT)),
        ],
        out_specs=pl.BlockSpec((2, BM, HALF), lambda i: (0, i, 0)),
        out_shape=jax.ShapeDtypeStruct((2, N, HALF), jnp.float32),
    )(h2, x2, w1t, b1row, w2t, b2row)


def _tc_final(x2, wft, bfrow):
    return pl.pallas_call(
        _final_body,
        grid=(GRID,),
        in_specs=[
            pl.BlockSpec((2, BM, HALF), lambda i: (0, i, 0)),
            _row_spec((FEAT, NOUT)),
            _row_spec((1, NOUT)),
        ],
        out_specs=pl.BlockSpec((BM, NOUT), lambda i: (i, 0)),
        out_shape=jax.ShapeDtypeStruct((N, NOUT), jnp.float32),
    )(x2, wft, bfrow)


def kernel(signals, edge_index, W0, b0, W1_0, b1_0, W2_0, b2_0, W1_1, b1_1,
           W2_1, W1_2, b1_2, W2_2, gn_gamma, gn_beta, gn_alpha, Wf, bf):
    src = edge_index[0].astype(jnp.int32)
    dst = edge_index[1].astype(jnp.int32)
    pad = EP - E
    srcp = jnp.concatenate([src, jnp.zeros((pad,), jnp.int32)]).reshape(NBLK, 1, EB)
    dstp = jnp.concatenate([dst, jnp.full((pad,), N, jnp.int32)]).reshape(NBLK, 1, EB)
    srcb2 = jnp.stack([srcp, srcp + N])

    sigp = jnp.concatenate([signals.reshape(N),
                            jnp.zeros((A1_ROWS - N,), jnp.float32)])

    w0row = W0.reshape(1, FEAT)
    b0row = b0.reshape(1, FEAT)
    grow = gn_gamma.reshape(1, FEAT)
    berow = gn_beta.reshape(1, FEAT)
    alrow = gn_alpha.reshape(1, FEAT)
    zrow = jnp.zeros((1, FEAT), jnp.float32)

    t1 = _agg1(sigp, srcp, dstp).reshape(A1_ROWS, 1)
    stats = _tc_stats(t1, w0row, b0row)
    x2 = _tc_norm(t1, stats, w0row, b0row, grow, berow, alrow)

    layer_params = [
        (W1_0.T, b1_0.reshape(1, FEAT), W2_0.T, b2_0.reshape(1, FEAT)),
        (W1_1.T, b1_1.reshape(1, FEAT), W2_1.T, zrow),
        (W1_2.T, b1_2.reshape(1, FEAT), W2_2.T, zrow),
    ]
    for w1t, b1row, w2t, b2row in layer_params:
        h2 = _agg128(x2.reshape(2 * N, HALF), srcb2, dstp)
        x2 = _tc_mlp(h2, x2, w1t, b1row, w2t, b2row)

    return _tc_final(x2, Wf.T, bf.reshape(1, NOUT))


# speedup vs baseline: 3.4703x; 1.2731x over previous
import dataclasses
import functools

import jax
import jax.numpy as jnp
from jax import lax
from jax.experimental import pallas as pl
from jax.experimental.pallas import tpu as pltpu
from jax.experimental.pallas import tpu_sc as plsc

N = 10000
FEAT = 256
HALF = 128
NOUT = 128
E = 160000

EB = 128
NBLK = 1280
EP = NBLK * EB
NSUB = 16
BPS = NBLK // NSUB
RPS = 624
RCH = 208
TAIL = N - NSUB * RPS
ACC_ROWS = N + 16
A1_ROWS = 10112
RED = 640

BM = 400
GRID = N // BM

_PREC = lax.Precision.HIGHEST

_mesh = plsc.VectorSubcoreMesh(core_axis_name="c", subcore_axis_name="s")

_sc_params = pltpu.CompilerParams()
if "needs_layout_passes" in pltpu.CompilerParams.__dataclass_fields__:
    _sc_params = dataclasses.replace(_sc_params, needs_layout_passes=False)


def _agg1_body(sigp_hbm, srcb_hbm, dstb_hbm, out_hbm,
               sig_v, idx3s, idx3d, acc_v, red_v, outacc, sh, sem):
    cid = lax.axis_index("c")
    sid = lax.axis_index("s")

    @pl.when(cid == 0)
    def _():
        pltpu.sync_copy(sigp_hbm, sig_v)
        pltpu.sync_copy(srcb_hbm.at[pl.ds(sid * BPS, BPS)], idx3s)
        pltpu.sync_copy(dstb_hbm.at[pl.ds(sid * BPS, BPS)], idx3d)

        @pl.loop(0, A1_ROWS // 16)
        def _(i):
            acc_v[pl.ds(i * 16, 16)] = jnp.zeros((16,), jnp.float32)

        @pl.loop(0, BPS)
        def _(j):
            @pl.loop(0, EB // 16)
            def _(k):
                sv = idx3s[j, 0, pl.ds(k * 16, 16)]
                dv = idx3d[j, 0, pl.ds(k * 16, 16)]
                vals = plsc.load_gather(sig_v, [sv])
                plsc.addupdate_scatter(acc_v, [dv], vals)

        pltpu.sync_copy(acc_v, sh.at[sid, 0])
        plsc.subcore_barrier()

        c0 = jnp.minimum(sid * RED, A1_ROWS - RED)
        pltpu.sync_copy(sh.at[:, :, pl.ds(c0, RED)], red_v)

        @pl.loop(0, RED // 16)
        def _(i):
            v = sig_v[pl.ds(c0 + i * 16, 16)]
            for k in range(NSUB):
                v = v + red_v[k, 0, pl.ds(i * 16, 16)]
            outacc[pl.ds(i * 16, 16)] = v

        pltpu.sync_copy(outacc, out_hbm.at[pl.ds(c0, RED)])


@functools.partial(
    pl.kernel,
    mesh=_mesh,
    out_type=jax.ShapeDtypeStruct((A1_ROWS,), jnp.float32),
    scratch_types=[
        pltpu.VMEM((A1_ROWS,), jnp.float32),
        pltpu.VMEM((BPS, 1, EB), jnp.int32),
        pltpu.VMEM((BPS, 1, EB), jnp.int32),
        pltpu.VMEM((A1_ROWS,), jnp.float32),
        pltpu.VMEM((NSUB, 1, RED), jnp.float32),
        pltpu.VMEM((RED,), jnp.float32),
        pltpu.VMEM_SHARED((NSUB, 1, A1_ROWS), jnp.float32),
        pltpu.SemaphoreType.DMA,
    ],
    compiler_params=_sc_params,
)
def _agg1(*args):
    _agg1_body(*args)


def _agg128_body(x_hbm, srcb2_hbm, dstb_hbm, out_hbm,
                 idx3s, idx3d, rows0, rows1, acc, semg0, semg1):
    cid = lax.axis_index("c")
    sid = lax.axis_index("s")
    off = cid * N

    hb = BPS // 2

    pltpu.sync_copy(srcb2_hbm.at[cid, pl.ds(sid * BPS, hb)], idx3s)
    pltpu.sync_copy(dstb_hbm.at[pl.ds(sid * BPS, hb)], idx3d)

    pltpu.sync_copy(x_hbm.at[pl.ds(off + sid * RPS, RPS)],
                    acc.at[pl.ds(sid * RPS, RPS)])

    @pl.when(sid == NSUB - 1)
    def _():
        pltpu.sync_copy(x_hbm.at[pl.ds(off + NSUB * RPS, TAIL)],
                        acc.at[pl.ds(NSUB * RPS, TAIL)])

    pltpu.make_async_copy(x_hbm.at[idx3s.at[0, 0]], rows0, semg0).start()

    plsc.subcore_barrier()

    for h in range(2):
        @pl.loop(0, hb // 2)
        def _(g):
            j = g * 2
            pltpu.make_async_copy(x_hbm.at[idx3s.at[j, 0]], rows0,
                                  semg0).wait()
            pltpu.make_async_copy(x_hbm.at[idx3s.at[j + 1, 0]], rows1,
                                  semg1).start()
            pltpu.sync_copy(rows0, acc.at[idx3d.at[j, 0]], add=True)
            pltpu.make_async_copy(x_hbm.at[idx3s.at[j + 1, 0]], rows1,
                                  semg1).wait()

            @pl.when(j + 2 < hb)
            def _():
                pltpu.make_async_copy(x_hbm.at[idx3s.at[j + 2, 0]], rows0,
                                      semg0).start()

            pltpu.sync_copy(rows1, acc.at[idx3d.at[j + 1, 0]], add=True)

        if h == 0:
            pltpu.sync_copy(srcb2_hbm.at[cid, pl.ds(sid * BPS + hb, hb)],
                            idx3s)
            pltpu.sync_copy(dstb_hbm.at[pl.ds(sid * BPS + hb, hb)], idx3d)
            pltpu.make_async_copy(x_hbm.at[idx3s.at[0, 0]], rows0,
                                  semg0).start()

    plsc.subcore_barrier()

    pltpu.sync_copy(acc.at[pl.ds(sid * RPS, RPS)],
                    out_hbm.at[cid, pl.ds(sid * RPS, RPS)])

    @pl.when(sid == NSUB - 1)
    def _():
        pltpu.sync_copy(acc.at[pl.ds(NSUB * RPS, TAIL)],
                        out_hbm.at[cid, pl.ds(NSUB * RPS, TAIL)])


@functools.partial(
    pl.kernel,
    mesh=_mesh,
    out_type=jax.ShapeDtypeStruct((2, N, HALF), jnp.float32),
    scratch_types=[
        pltpu.VMEM((BPS // 2, 1, EB), jnp.int32),
        pltpu.VMEM((BPS // 2, 1, EB), jnp.int32),
        pltpu.VMEM((EB, HALF), jnp.float32),
        pltpu.VMEM((EB, HALF), jnp.float32),
        pltpu.VMEM_SHARED((ACC_ROWS, HALF), jnp.float32),
        pltpu.SemaphoreType.DMA,
        pltpu.SemaphoreType.DMA,
    ],
)
def _agg128(*args):
    _agg128_body(*args)


def _stats_body(t_ref, w0_ref, b0_ref, stats_ref):
    i = pl.program_id(0)
    xp = jnp.maximum(t_ref[...] * w0_ref[...] + b0_ref[...], 0.0)

    @pl.when(i == 0)
    def _():
        stats_ref[...] = jnp.zeros_like(stats_ref)

    stats_ref[0:1, :] += jnp.sum(xp, axis=0, keepdims=True)
    stats_ref[1:2, :] += jnp.sum(xp * xp, axis=0, keepdims=True)


def _norm_body(t_ref, stats_ref, w0_ref, b0_ref, g_ref, be_ref, al_ref, out_ref):
    xp = jnp.maximum(t_ref[...] * w0_ref[...] + b0_ref[...], 0.0)
    m = stats_ref[0:1, :] * (1.0 / N)
    ex2 = stats_ref[1:2, :] * (1.0 / N)
    al = al_ref[...]
    var = ex2 - 2.0 * al * m * m + al * al * m * m
    y = g_ref[...] * ((xp - al * m) * lax.rsqrt(var + 1e-5)) + be_ref[...]
    out_ref[0] = y[:, :HALF]
    out_ref[1] = y[:, HALF:]


def _mlp_body(h_ref, x_ref, w1_ref, b1_ref, w2_ref, b2_ref, out_ref):
    h = jnp.concatenate([h_ref[0], h_ref[1]], axis=1)
    z = jnp.maximum(
        jnp.dot(h, w1_ref[...], preferred_element_type=jnp.float32,
                precision=_PREC) + b1_ref[...], 0.0)
    y = jnp.dot(z, w2_ref[...], preferred_element_type=jnp.float32,
                precision=_PREC) + b2_ref[...]
    x = jnp.concatenate([x_ref[0], x_ref[1]], axis=1)
    xn = x + jnp.maximum(y, 0.0)
    out_ref[0] = xn[:, :HALF]
    out_ref[1] = xn[:, HALF:]


def _final_body(x_ref, wf_ref, bf_ref, out_ref):
    x = jnp.concatenate([x_ref[0], x_ref[1]], axis=1)
    out_ref[...] = jnp.dot(x, wf_ref[...], preferred_element_type=jnp.float32,
                           precision=_PREC) + bf_ref[...]


def _row_spec(shape):
    return pl.BlockSpec(shape, lambda i: tuple(0 for _ in shape))


def _tc_stats(t1, w0row, b0row):
    return pl.pallas_call(
        _stats_body,
        grid=(GRID,),
        in_specs=[
            pl.BlockSpec((BM, 1), lambda i: (i, 0)),
            _row_spec((1, FEAT)),
            _row_spec((1, FEAT)),
        ],
        out_specs=pl.BlockSpec((8, FEAT), lambda i: (0, 0)),
        out_shape=jax.ShapeDtypeStruct((8, FEAT), jnp.float32),
    )(t1, w0row, b0row)


def _tc_norm(t1, stats, w0row, b0row, grow, berow, alrow):
    return pl.pallas_call(
        _norm_body,
        grid=(GRID,),
        in_specs=[
            pl.BlockSpec((BM, 1), lambda i: (i, 0)),
            _row_spec((8, FEAT)),
            _row_spec((1, FEAT)),
            _row_spec((1, FEAT)),
            _row_spec((1, FEAT)),
            _row_spec((1, FEAT)),
            _row_spec((1, FEAT)),
        ],
        out_specs=pl.BlockSpec((2, BM, HALF), lambda i: (0, i, 0)),
        out_shape=jax.ShapeDtypeStruct((2, N, HALF), jnp.float32),
    )(t1, stats, w0row, b0row, grow, berow, alrow)


def _tc_mlp(h2, x2, w1t, b1row, w2t, b2row):
    return pl.pallas_call(
        _mlp_body,
        grid=(GRID,),
        in_specs=[
            pl.BlockSpec((2, BM, HALF), lambda i: (0, i, 0)),
            pl.BlockSpec((2, BM, HALF), lambda i: (0, i, 0)),
            _row_spec((FEAT, FEAT)),
            _row_spec((1, FEAT)),
            _row_spec((FEAT, FEAT)),
            _row_spec((1, FEAT)),
        ],
        out_specs=pl.BlockSpec((2, BM, HALF), lambda i: (0, i, 0)),
        out_shape=jax.ShapeDtypeStruct((2, N, HALF), jnp.float32),
    )(h2, x2, w1t, b1row, w2t, b2row)


def _tc_final(x2, wft, bfrow):
    return pl.pallas_call(
        _final_body,
        grid=(GRID,),
        in_specs=[
            pl.BlockSpec((2, BM, HALF), lambda i: (0, i, 0)),
            _row_spec((FEAT, NOUT)),
            _row_spec((1, NOUT)),
        ],
        out_specs=pl.BlockSpec((BM, NOUT), lambda i: (i, 0)),
        out_shape=jax.ShapeDtypeStruct((N, NOUT), jnp.float32),
    )(x2, wft, bfrow)


def kernel(signals, edge_index, W0, b0, W1_0, b1_0, W2_0, b2_0, W1_1, b1_1,
           W2_1, W1_2, b1_2, W2_2, gn_gamma, gn_beta, gn_alpha, Wf, bf):
    src = edge_index[0].astype(jnp.int32)
    dst = edge_index[1].astype(jnp.int32)
    pad = EP - E
    srcp = jnp.concatenate([src, jnp.zeros((pad,), jnp.int32)]).reshape(NBLK, 1, EB)
    dstp = jnp.concatenate([dst, jnp.full((pad,), N, jnp.int32)]).reshape(NBLK, 1, EB)
    srcb2 = jnp.stack([srcp, srcp + N])

    sigp = jnp.concatenate([signals.reshape(N),
                            jnp.zeros((A1_ROWS - N,), jnp.float32)])

    w0row = W0.reshape(1, FEAT)
    b0row = b0.reshape(1, FEAT)
    grow = gn_gamma.reshape(1, FEAT)
    berow = gn_beta.reshape(1, FEAT)
    alrow = gn_alpha.reshape(1, FEAT)
    zrow = jnp.zeros((1, FEAT), jnp.float32)

    t1 = _agg1(sigp, srcp, dstp).reshape(A1_ROWS, 1)
    stats = _tc_stats(t1, w0row, b0row)
    x2 = _tc_norm(t1, stats, w0row, b0row, grow, berow, alrow)

    layer_params = [
        (W1_0.T, b1_0.reshape(1, FEAT), W2_0.T, b2_0.reshape(1, FEAT)),
        (W1_1.T, b1_1.reshape(1, FEAT), W2_1.T, zrow),
        (W1_2.T, b1_2.reshape(1, FEAT), W2_2.T, zrow),
    ]
    for w1t, b1row, w2t, b2row in layer_params:
        h2 = _agg128(x2.reshape(2 * N, HALF), srcb2, dstp)
        x2 = _tc_mlp(h2, x2, w1t, b1row, w2t, b2row)

    return _tc_final(x2, Wf.T, bf.reshape(1, NOUT))
